# baseline (speedup 1.0000x reference)
import functools

import jax
import jax.numpy as jnp
from jax import lax
from jax.experimental import pallas as pl
from jax.experimental.pallas import tpu as pltpu
from jax.experimental.pallas import tpu_sc as plsc

R = 500
H = 256
N = 10000
E = 160000
R_PAD = 512

NC = 2
NS = 16
NW = NC * NS
K = 112
NB = -(-E // (NW * K))
E_PAD = NW * NB * K
B = 512
NBC = (20, 20, 5)
RW = 256


def _make_sc_gather(nbc):
    mesh = plsc.VectorSubcoreMesh(core_axis_name="c", subcore_axis_name="s")

    @functools.partial(
        pl.kernel,
        mesh=mesh,
        out_type=jax.ShapeDtypeStruct((NW, nbc, K, H // 2), jnp.int32),
        scratch_types=[
            pltpu.VMEM((nbc, K), jnp.int32),
            pltpu.VMEM((K, H // 2), jnp.int32),
            pltpu.VMEM((K, H // 2), jnp.int32),
            pltpu.VMEM((K, H // 2), jnp.int32),
            pltpu.VMEM((K, H // 2), jnp.int32),
            pltpu.SemaphoreType.DMA,
            pltpu.SemaphoreType.DMA,
            pltpu.SemaphoreType.DMA,
            pltpu.SemaphoreType.DMA,
            pltpu.SemaphoreType.DMA,
            pltpu.SemaphoreType.DMA,
            pltpu.SemaphoreType.DMA,
            pltpu.SemaphoreType.DMA,
        ],
    )
    def gatherk(ent_hbm, idx_hbm, rows_out, idx_all, rows0, rows1, rows2,
                rows3, gsem0, gsem1, gsem2, gsem3, wsem0, wsem1, wsem2,
                wsem3):
        c = lax.axis_index("c")
        s = lax.axis_index("s")
        wid = s * NC + c

        pltpu.sync_copy(idx_hbm.at[wid], idx_all)

        ND = 4
        rows = (rows0, rows1, rows2, rows3)
        gsem = (gsem0, gsem1, gsem2, gsem3)
        wsem = (wsem0, wsem1, wsem2, wsem3)
        writes = [None] * ND
        gets = [None] * ND
        for j in range(nbc + ND - 1):
            if j < nbc:
                b = j % ND
                if writes[b] is not None:
                    writes[b].wait()
                gets[b] = pltpu.async_copy(ent_hbm.at[idx_all.at[j]],
                                           rows[b], gsem[b])
            if j >= ND - 1:
                jd = j - ND + 1
                pb = jd % ND
                gets[pb].wait()
                writes[pb] = pltpu.async_copy(rows[pb],
                                              rows_out.at[wid, jd], wsem[pb])
        for b in range(ND):
            writes[b].wait()

    return gatherk


_GATHER16 = _make_sc_gather(NBC[0])
_GATHER8 = _make_sc_gather(NBC[2])


def _accum_block(seg_ref, gath_ref, acc, cnt):
    seg = seg_ref[0]
    x = gath_ref[0]
    lo = lax.bitcast_convert_type(x << 16, jnp.float32).astype(jnp.bfloat16)
    hi = lax.bitcast_convert_type(
        x & jnp.int32(-65536), jnp.float32).astype(jnp.bfloat16)
    rows = jnp.concatenate([lo, hi], axis=1)
    dn = (((1,), (0,)), ((), ()))
    ones_b = jnp.ones((B, 128), jnp.bfloat16)
    mn = jnp.min(seg)
    mx = jnp.max(seg)
    base = jnp.minimum((mn // 8) * 8, R_PAD - RW)
    narrow = mx < base + RW

    @pl.when(narrow)
    def _narrow():
        m = (lax.broadcasted_iota(jnp.int32, (RW, B), 0) + base) == seg
        ohb = jnp.where(m, 1.0, 0.0).astype(jnp.bfloat16)
        acc[pl.ds(base, RW), :] += lax.dot_general(
            ohb, rows, dn, preferred_element_type=jnp.float32)
        cnt[pl.ds(base, RW), :] += lax.dot_general(
            ohb, ones_b, dn, preferred_element_type=jnp.float32)[:, :1]

    @pl.when(jnp.logical_not(narrow))
    def _full():
        m = lax.broadcasted_iota(jnp.int32, (R_PAD, B), 0) == seg
        ohb = jnp.where(m, 1.0, 0.0).astype(jnp.bfloat16)
        acc[...] += lax.dot_general(ohb, rows, dn,
                                    preferred_element_type=jnp.float32)
        cnt[...] += lax.dot_general(ohb, ones_b, dn,
                                    preferred_element_type=jnp.float32)[:, :1]


def _make_segmid(nblk):
    def body(seg_ref, gath_ref, sums0_ref, cnt0_ref, sums_ref, cnt_ref,
             acc, cnt):
        i = pl.program_id(0)

        @pl.when(i == 0)
        def _init():
            acc[...] = sums0_ref[...]
            cnt[...] = cnt0_ref[...]

        _accum_block(seg_ref, gath_ref, acc, cnt)

        @pl.when(i == nblk - 1)
        def _fin():
            sums_ref[...] = acc[...]
            cnt_ref[...] = cnt[...]

    _full_ = lambda i: (0, 0)
    return pl.pallas_call(
        body,
        grid=(nblk,),
        in_specs=[
            pl.BlockSpec((1, 1, B), lambda i: (i, 0, 0)),
            pl.BlockSpec((1, B, H // 2), lambda i: (i, 0, 0)),
            pl.BlockSpec((R_PAD, H), _full_),
            pl.BlockSpec((R_PAD, 1), _full_),
        ],
        out_specs=[
            pl.BlockSpec((R_PAD, H), _full_),
            pl.BlockSpec((R_PAD, 1), _full_),
        ],
        out_shape=[
            jax.ShapeDtypeStruct((R_PAD, H), jnp.float32),
            jax.ShapeDtypeStruct((R_PAD, 1), jnp.float32),
        ],
        scratch_shapes=[
            pltpu.VMEM((R_PAD, H), jnp.float32),
            pltpu.VMEM((R_PAD, 1), jnp.float32),
        ],
    )


def _make_segfin(nblk):
    def body(seg_ref, gath_ref, sums0_ref, cnt0_ref, rel_ref, w1_ref,
             w2p_ref, whh_ref, bih_ref, bhh_ref, out_ref, acc, cnt):
        i = pl.program_id(0)

        @pl.when(i == 0)
        def _init():
            acc[...] = sums0_ref[...]
            cnt[...] = cnt0_ref[...]

        _accum_block(seg_ref, gath_ref, acc, cnt)

        @pl.when(i == nblk - 1)
        def _epilogue():
            sums = acc[...]
            mean = sums / jnp.maximum(cnt[...], 1.0)
            rel = rel_ref[...]
            whh = whh_ref[...]
            dnt = (((1,), (1,)), ((), ()))
            gi = lax.dot_general(rel, w1_ref[...], dnt,
                                 preferred_element_type=jnp.float32)
            gi = gi + lax.dot_general(mean, w2p_ref[...], dnt,
                                      preferred_element_type=jnp.float32)
            gi = gi + bih_ref[...]
            gh = lax.dot_general(rel, whh, dnt,
                                 preferred_element_type=jnp.float32)
            gh = gh + bhh_ref[...]
            r = jax.nn.sigmoid(gi[:, :H] + gh[:, :H])
            z = jax.nn.sigmoid(gi[:, H:2 * H] + gh[:, H:2 * H])
            n = jnp.tanh(gi[:, 2 * H:] + r * gh[:, 2 * H:])
            h0 = (1.0 - z) * n + z * rel
            norm = jnp.sqrt(jnp.sum(h0 * h0, axis=1, keepdims=True))
            out_ref[...] = h0 / jnp.maximum(norm, 1e-12)

    _full_ = lambda i: (0, 0)
    return pl.pallas_call(
        body,
        grid=(nblk,),
        in_specs=[
            pl.BlockSpec((1, 1, B), lambda i: (i, 0, 0)),
            pl.BlockSpec((1, B, H // 2), lambda i: (i, 0, 0)),
            pl.BlockSpec((R_PAD, H), _full_),
            pl.BlockSpec((R_PAD, 1), _full_),
            pl.BlockSpec((R_PAD, H), _full_),
            pl.BlockSpec((3 * H, H), _full_),
            pl.BlockSpec((3 * H, H), _full_),
            pl.BlockSpec((3 * H, H), _full_),
            pl.BlockSpec((1, 3 * H), _full_),
            pl.BlockSpec((1, 3 * H), _full_),
        ],
        out_specs=pl.BlockSpec((R_PAD, H), _full_),
        out_shape=jax.ShapeDtypeStruct((R_PAD, H), jnp.float32),
        scratch_shapes=[
            pltpu.VMEM((R_PAD, H), jnp.float32),
            pltpu.VMEM((R_PAD, 1), jnp.float32),
        ],
    )


def _nblk(nbc):
    return NW * nbc * K // B


_SEGMID16 = _make_segmid(_nblk(NBC[0]))
_SEGFIN8 = _make_segfin(_nblk(NBC[2]))


def kernel(rel_embs, ent_embs, r_to_e_flat, seg_ids, e_r_bias, num_rels,
           W_ih, W_hh, b_ih, b_hh):
    pad = E_PAD - E
    idx_rs = jnp.concatenate(
        [r_to_e_flat.astype(jnp.int32),
         jnp.zeros((pad,), jnp.int32)]).reshape(NW, NB, K)
    seg_rs = jnp.concatenate(
        [seg_ids.astype(jnp.int32),
         jnp.full((pad,), R_PAD - 1, jnp.int32)]).reshape(NW, NB, K)

    b0, b1 = NBC[0], NBC[0] + NBC[1]
    idx_c = (idx_rs[:, :b0], idx_rs[:, b0:b1], idx_rs[:, b1:])
    seg_c = (seg_rs[:, :b0].reshape(_nblk(NBC[0]), 1, B),
             seg_rs[:, b0:b1].reshape(_nblk(NBC[1]), 1, B),
             seg_rs[:, b1:].reshape(_nblk(NBC[2]), 1, B))

    ent_i32 = lax.bitcast_convert_type(
        ent_embs.astype(jnp.bfloat16).reshape(N, H // 2, 2), jnp.int32)
    g1 = _GATHER16(ent_i32, idx_c[0])
    g2 = _GATHER16(ent_i32, idx_c[1])
    g3 = _GATHER8(ent_i32, idx_c[2])

    z_s = jnp.zeros((R_PAD, H), jnp.float32)
    z_c = jnp.zeros((R_PAD, 1), jnp.float32)
    s1, c1 = _SEGMID16(seg_c[0], g1.reshape(_nblk(NBC[0]), B, H // 2),
                       z_s, z_c)
    s2, c2 = _SEGMID16(seg_c[1], g2.reshape(_nblk(NBC[1]), B, H // 2),
                       s1, c1)
    rel_pad = jnp.pad(rel_embs, ((0, R_PAD - R), (0, 0)))
    w2 = W_ih[:, H:]
    w2p = jnp.concatenate([w2[:, 0::2], w2[:, 1::2]], axis=1)
    out = _SEGFIN8(seg_c[2], g3.reshape(_nblk(NBC[2]), B, H // 2), s2, c2,
                   rel_pad, W_ih[:, :H], w2p, W_hh,
                   b_ih.reshape(1, 3 * H), b_hh.reshape(1, 3 * H))
    return out[:R]


# speedup vs baseline: 1.1285x; 1.1285x over previous
import functools

import jax
import jax.numpy as jnp
from jax import lax
from jax.experimental import pallas as pl
from jax.experimental.pallas import tpu as pltpu
from jax.experimental.pallas import tpu_sc as plsc

R = 500
H = 256
N = 10000
E = 160000
R_PAD = 512

NC = 2
NS = 16
NW = NC * NS
K = 112
NB = -(-E // (NW * K))
E_PAD = NW * NB * K
B = 512
NBC = (20, 20, 5)
RW = 256


def _make_sc_gather(nbc):
    mesh = plsc.VectorSubcoreMesh(core_axis_name="c", subcore_axis_name="s")

    @functools.partial(
        pl.kernel,
        mesh=mesh,
        out_type=jax.ShapeDtypeStruct((NW, nbc, K, H), jnp.float32),
        scratch_types=[
            pltpu.VMEM((nbc, K), jnp.int32),
            pltpu.VMEM((K, H), jnp.float32),
            pltpu.VMEM((K, H), jnp.float32),
            pltpu.VMEM((K, H), jnp.float32),
            pltpu.VMEM((K, H), jnp.float32),
            pltpu.SemaphoreType.DMA,
            pltpu.SemaphoreType.DMA,
            pltpu.SemaphoreType.DMA,
            pltpu.SemaphoreType.DMA,
            pltpu.SemaphoreType.DMA,
            pltpu.SemaphoreType.DMA,
            pltpu.SemaphoreType.DMA,
            pltpu.SemaphoreType.DMA,
        ],
    )
    def gatherk(ent_hbm, idx_hbm, rows_out, idx_all, rows0, rows1, rows2,
                rows3, gsem0, gsem1, gsem2, gsem3, wsem0, wsem1, wsem2,
                wsem3):
        c = lax.axis_index("c")
        s = lax.axis_index("s")
        wid = s * NC + c

        pltpu.sync_copy(idx_hbm.at[wid], idx_all)

        ND = 4
        rows = (rows0, rows1, rows2, rows3)
        gsem = (gsem0, gsem1, gsem2, gsem3)
        wsem = (wsem0, wsem1, wsem2, wsem3)
        writes = [None] * ND
        gets = [None] * ND
        for j in range(nbc + ND - 1):
            if j < nbc:
                b = j % ND
                if writes[b] is not None:
                    writes[b].wait()
                gets[b] = pltpu.async_copy(ent_hbm.at[idx_all.at[j]],
                                           rows[b], gsem[b])
            if j >= ND - 1:
                jd = j - ND + 1
                pb = jd % ND
                gets[pb].wait()
                writes[pb] = pltpu.async_copy(rows[pb],
                                              rows_out.at[wid, jd], wsem[pb])
        for b in range(ND):
            writes[b].wait()

    return gatherk


_GATHER16 = _make_sc_gather(NBC[0])
_GATHER8 = _make_sc_gather(NBC[2])


def _accum_block(seg_ref, gath_ref, acc, cnt):
    seg = seg_ref[0]
    rows = gath_ref[0].astype(jnp.bfloat16)
    dn = (((1,), (0,)), ((), ()))
    ones_b = jnp.ones((B, 128), jnp.bfloat16)
    mn = jnp.min(seg)
    mx = jnp.max(seg)
    base = jnp.minimum((mn // 8) * 8, R_PAD - RW)
    narrow = mx < base + RW

    @pl.when(narrow)
    def _narrow():
        m = (lax.broadcasted_iota(jnp.int32, (RW, B), 0) + base) == seg
        ohb = jnp.where(m, 1.0, 0.0).astype(jnp.bfloat16)
        acc[pl.ds(base, RW), :] += lax.dot_general(
            ohb, rows, dn, preferred_element_type=jnp.float32)
        cnt[pl.ds(base, RW), :] += lax.dot_general(
            ohb, ones_b, dn, preferred_element_type=jnp.float32)[:, :1]

    @pl.when(jnp.logical_not(narrow))
    def _full():
        m = lax.broadcasted_iota(jnp.int32, (R_PAD, B), 0) == seg
        ohb = jnp.where(m, 1.0, 0.0).astype(jnp.bfloat16)
        acc[...] += lax.dot_general(ohb, rows, dn,
                                    preferred_element_type=jnp.float32)
        cnt[...] += lax.dot_general(ohb, ones_b, dn,
                                    preferred_element_type=jnp.float32)[:, :1]


def _make_segmid(nblk):
    def body(seg_ref, gath_ref, sums0_ref, cnt0_ref, sums_ref, cnt_ref,
             acc, cnt):
        i = pl.program_id(0)

        @pl.when(i == 0)
        def _init():
            acc[...] = sums0_ref[...]
            cnt[...] = cnt0_ref[...]

        _accum_block(seg_ref, gath_ref, acc, cnt)

        @pl.when(i == nblk - 1)
        def _fin():
            sums_ref[...] = acc[...]
            cnt_ref[...] = cnt[...]

    _full_ = lambda i: (0, 0)
    return pl.pallas_call(
        body,
        grid=(nblk,),
        in_specs=[
            pl.BlockSpec((1, 1, B), lambda i: (i, 0, 0)),
            pl.BlockSpec((1, B, H), lambda i: (i, 0, 0)),
            pl.BlockSpec((R_PAD, H), _full_),
            pl.BlockSpec((R_PAD, 1), _full_),
        ],
        out_specs=[
            pl.BlockSpec((R_PAD, H), _full_),
            pl.BlockSpec((R_PAD, 1), _full_),
        ],
        out_shape=[
            jax.ShapeDtypeStruct((R_PAD, H), jnp.float32),
            jax.ShapeDtypeStruct((R_PAD, 1), jnp.float32),
        ],
        scratch_shapes=[
            pltpu.VMEM((R_PAD, H), jnp.float32),
            pltpu.VMEM((R_PAD, 1), jnp.float32),
        ],
    )


def _make_segfin(nblk):
    def body(seg_ref, gath_ref, sums0_ref, cnt0_ref, rel_ref, wih_ref,
             whh_ref, bih_ref, bhh_ref, out_ref, acc, cnt):
        i = pl.program_id(0)

        @pl.when(i == 0)
        def _init():
            acc[...] = sums0_ref[...]
            cnt[...] = cnt0_ref[...]

        _accum_block(seg_ref, gath_ref, acc, cnt)

        @pl.when(i == nblk - 1)
        def _epilogue():
            sums = acc[...]
            mean = sums / jnp.maximum(cnt[...], 1.0)
            rel = rel_ref[...]
            wih = wih_ref[...]
            whh = whh_ref[...]
            dnt = (((1,), (1,)), ((), ()))
            gi = lax.dot_general(rel, wih[:, :H], dnt,
                                 preferred_element_type=jnp.float32)
            gi = gi + lax.dot_general(mean, wih[:, H:], dnt,
                                      preferred_element_type=jnp.float32)
            gi = gi + bih_ref[...]
            gh = lax.dot_general(rel, whh, dnt,
                                 preferred_element_type=jnp.float32)
            gh = gh + bhh_ref[...]
            r = jax.nn.sigmoid(gi[:, :H] + gh[:, :H])
            z = jax.nn.sigmoid(gi[:, H:2 * H] + gh[:, H:2 * H])
            n = jnp.tanh(gi[:, 2 * H:] + r * gh[:, 2 * H:])
            h0 = (1.0 - z) * n + z * rel
            norm = jnp.sqrt(jnp.sum(h0 * h0, axis=1, keepdims=True))
            out_ref[...] = h0 / jnp.maximum(norm, 1e-12)

    _full_ = lambda i: (0, 0)
    return pl.pallas_call(
        body,
        grid=(nblk,),
        in_specs=[
            pl.BlockSpec((1, 1, B), lambda i: (i, 0, 0)),
            pl.BlockSpec((1, B, H), lambda i: (i, 0, 0)),
            pl.BlockSpec((R_PAD, H), _full_),
            pl.BlockSpec((R_PAD, 1), _full_),
            pl.BlockSpec((R_PAD, H), _full_),
            pl.BlockSpec((3 * H, 2 * H), _full_),
            pl.BlockSpec((3 * H, H), _full_),
            pl.BlockSpec((1, 3 * H), _full_),
            pl.BlockSpec((1, 3 * H), _full_),
        ],
        out_specs=pl.BlockSpec((R_PAD, H), _full_),
        out_shape=jax.ShapeDtypeStruct((R_PAD, H), jnp.float32),
        scratch_shapes=[
            pltpu.VMEM((R_PAD, H), jnp.float32),
            pltpu.VMEM((R_PAD, 1), jnp.float32),
        ],
    )


def _nblk(nbc):
    return NW * nbc * K // B


_SEGMID16 = _make_segmid(_nblk(NBC[0]))
_SEGFIN8 = _make_segfin(_nblk(NBC[2]))


def kernel(rel_embs, ent_embs, r_to_e_flat, seg_ids, e_r_bias, num_rels,
           W_ih, W_hh, b_ih, b_hh):
    pad = E_PAD - E
    idx_rs = jnp.concatenate(
        [r_to_e_flat.astype(jnp.int32),
         jnp.zeros((pad,), jnp.int32)]).reshape(NW, NB, K)
    seg_rs = jnp.concatenate(
        [seg_ids.astype(jnp.int32),
         jnp.full((pad,), R_PAD - 1, jnp.int32)]).reshape(NW, NB, K)

    b0, b1 = NBC[0], NBC[0] + NBC[1]
    idx_c = (idx_rs[:, :b0], idx_rs[:, b0:b1], idx_rs[:, b1:])
    seg_c = (seg_rs[:, :b0].reshape(_nblk(NBC[0]), 1, B),
             seg_rs[:, b0:b1].reshape(_nblk(NBC[1]), 1, B),
             seg_rs[:, b1:].reshape(_nblk(NBC[2]), 1, B))

    g1 = _GATHER16(ent_embs, idx_c[0])
    g2 = _GATHER16(ent_embs, idx_c[1])
    g3 = _GATHER8(ent_embs, idx_c[2])

    z_s = jnp.zeros((R_PAD, H), jnp.float32)
    z_c = jnp.zeros((R_PAD, 1), jnp.float32)
    s1, c1 = _SEGMID16(seg_c[0], g1.reshape(_nblk(NBC[0]), B, H), z_s, z_c)
    s2, c2 = _SEGMID16(seg_c[1], g2.reshape(_nblk(NBC[1]), B, H), s1, c1)
    rel_pad = jnp.pad(rel_embs, ((0, R_PAD - R), (0, 0)))
    out = _SEGFIN8(seg_c[2], g3.reshape(_nblk(NBC[2]), B, H), s2, c2,
                   rel_pad, W_ih, W_hh, b_ih.reshape(1, 3 * H),
                   b_hh.reshape(1, 3 * H))
    return out[:R]


# speedup vs baseline: 1.1593x; 1.0273x over previous
import functools

import jax
import jax.numpy as jnp
from jax import lax
from jax.experimental import pallas as pl
from jax.experimental.pallas import tpu as pltpu
from jax.experimental.pallas import tpu_sc as plsc

R = 500
H = 256
N = 10000
E = 160000
R_PAD = 512

NC = 2
NS = 16
NW = NC * NS
K = 112
NB = -(-E // (NW * K))
E_PAD = NW * NB * K
B = 512
NBC = (14, 14, 14, 3)
RW = 256


def _make_sc_gather(nbc):
    mesh = plsc.VectorSubcoreMesh(core_axis_name="c", subcore_axis_name="s")

    @functools.partial(
        pl.kernel,
        mesh=mesh,
        out_type=jax.ShapeDtypeStruct((NW, nbc, K, H), jnp.float32),
        scratch_types=[
            pltpu.VMEM((nbc, K), jnp.int32),
            pltpu.VMEM((K, H), jnp.float32),
            pltpu.VMEM((K, H), jnp.float32),
            pltpu.VMEM((K, H), jnp.float32),
            pltpu.VMEM((K, H), jnp.float32),
            pltpu.SemaphoreType.DMA,
            pltpu.SemaphoreType.DMA,
            pltpu.SemaphoreType.DMA,
            pltpu.SemaphoreType.DMA,
            pltpu.SemaphoreType.DMA,
            pltpu.SemaphoreType.DMA,
            pltpu.SemaphoreType.DMA,
            pltpu.SemaphoreType.DMA,
        ],
    )
    def gatherk(ent_hbm, idx_hbm, rows_out, idx_all, rows0, rows1, rows2,
                rows3, gsem0, gsem1, gsem2, gsem3, wsem0, wsem1, wsem2,
                wsem3):
        c = lax.axis_index("c")
        s = lax.axis_index("s")
        wid = s * NC + c

        pltpu.sync_copy(idx_hbm.at[wid], idx_all)

        ND = 4
        rows = (rows0, rows1, rows2, rows3)
        gsem = (gsem0, gsem1, gsem2, gsem3)
        wsem = (wsem0, wsem1, wsem2, wsem3)
        writes = [None] * ND
        gets = [None] * ND
        for j in range(nbc + ND - 1):
            if j < nbc:
                b = j % ND
                if writes[b] is not None:
                    writes[b].wait()
                gets[b] = pltpu.async_copy(ent_hbm.at[idx_all.at[j]],
                                           rows[b], gsem[b])
            if j >= ND - 1:
                jd = j - ND + 1
                pb = jd % ND
                gets[pb].wait()
                writes[pb] = pltpu.async_copy(rows[pb],
                                              rows_out.at[wid, jd], wsem[pb])
        for b in range(ND):
            if writes[b] is not None:
                writes[b].wait()

    return gatherk


_GATHER14 = _make_sc_gather(NBC[0])
_GATHER3 = _make_sc_gather(NBC[3])


def _accum_block(seg_ref, gath_ref, acc, cnt):
    seg = seg_ref[0]
    rows = gath_ref[0].astype(jnp.bfloat16)
    dn = (((1,), (0,)), ((), ()))
    ones_b = jnp.ones((B, 128), jnp.bfloat16)
    mn = jnp.min(seg)
    mx = jnp.max(seg)
    base = jnp.minimum((mn // 8) * 8, R_PAD - RW)
    narrow = mx < base + RW

    @pl.when(narrow)
    def _narrow():
        m = (lax.broadcasted_iota(jnp.int32, (RW, B), 0) + base) == seg
        ohb = jnp.where(m, 1.0, 0.0).astype(jnp.bfloat16)
        acc[pl.ds(base, RW), :] += lax.dot_general(
            ohb, rows, dn, preferred_element_type=jnp.float32)
        cnt[pl.ds(base, RW), :] += lax.dot_general(
            ohb, ones_b, dn, preferred_element_type=jnp.float32)[:, :1]

    @pl.when(jnp.logical_not(narrow))
    def _full():
        m = lax.broadcasted_iota(jnp.int32, (R_PAD, B), 0) == seg
        ohb = jnp.where(m, 1.0, 0.0).astype(jnp.bfloat16)
        acc[...] += lax.dot_general(ohb, rows, dn,
                                    preferred_element_type=jnp.float32)
        cnt[...] += lax.dot_general(ohb, ones_b, dn,
                                    preferred_element_type=jnp.float32)[:, :1]


def _make_segmid(nblk):
    def body(seg_ref, gath_ref, sums0_ref, cnt0_ref, sums_ref, cnt_ref,
             acc, cnt):
        i = pl.program_id(0)

        @pl.when(i == 0)
        def _init():
            acc[...] = sums0_ref[...]
            cnt[...] = cnt0_ref[...]

        _accum_block(seg_ref, gath_ref, acc, cnt)

        @pl.when(i == nblk - 1)
        def _fin():
            sums_ref[...] = acc[...]
            cnt_ref[...] = cnt[...]

    _full_ = lambda i: (0, 0)
    return pl.pallas_call(
        body,
        grid=(nblk,),
        in_specs=[
            pl.BlockSpec((1, 1, B), lambda i: (i, 0, 0)),
            pl.BlockSpec((1, B, H), lambda i: (i, 0, 0)),
            pl.BlockSpec((R_PAD, H), _full_),
            pl.BlockSpec((R_PAD, 1), _full_),
        ],
        out_specs=[
            pl.BlockSpec((R_PAD, H), _full_),
            pl.BlockSpec((R_PAD, 1), _full_),
        ],
        out_shape=[
            jax.ShapeDtypeStruct((R_PAD, H), jnp.float32),
            jax.ShapeDtypeStruct((R_PAD, 1), jnp.float32),
        ],
        scratch_shapes=[
            pltpu.VMEM((R_PAD, H), jnp.float32),
            pltpu.VMEM((R_PAD, 1), jnp.float32),
        ],
    )


def _make_segfin(nblk):
    def body(seg_ref, gath_ref, sums0_ref, cnt0_ref, rel_ref, wih_ref,
             whh_ref, bih_ref, bhh_ref, out_ref, acc, cnt):
        i = pl.program_id(0)

        @pl.when(i == 0)
        def _init():
            acc[...] = sums0_ref[...]
            cnt[...] = cnt0_ref[...]

        _accum_block(seg_ref, gath_ref, acc, cnt)

        @pl.when(i == nblk - 1)
        def _epilogue():
            sums = acc[...]
            mean = sums / jnp.maximum(cnt[...], 1.0)
            rel = rel_ref[...]
            wih = wih_ref[...]
            whh = whh_ref[...]
            dnt = (((1,), (1,)), ((), ()))
            gi = lax.dot_general(rel, wih[:, :H], dnt,
                                 preferred_element_type=jnp.float32)
            gi = gi + lax.dot_general(mean, wih[:, H:], dnt,
                                      preferred_element_type=jnp.float32)
            gi = gi + bih_ref[...]
            gh = lax.dot_general(rel, whh, dnt,
                                 preferred_element_type=jnp.float32)
            gh = gh + bhh_ref[...]
            r = jax.nn.sigmoid(gi[:, :H] + gh[:, :H])
            z = jax.nn.sigmoid(gi[:, H:2 * H] + gh[:, H:2 * H])
            n = jnp.tanh(gi[:, 2 * H:] + r * gh[:, 2 * H:])
            h0 = (1.0 - z) * n + z * rel
            norm = jnp.sqrt(jnp.sum(h0 * h0, axis=1, keepdims=True))
            out_ref[...] = h0 / jnp.maximum(norm, 1e-12)

    _full_ = lambda i: (0, 0)
    return pl.pallas_call(
        body,
        grid=(nblk,),
        in_specs=[
            pl.BlockSpec((1, 1, B), lambda i: (i, 0, 0)),
            pl.BlockSpec((1, B, H), lambda i: (i, 0, 0)),
            pl.BlockSpec((R_PAD, H), _full_),
            pl.BlockSpec((R_PAD, 1), _full_),
            pl.BlockSpec((R_PAD, H), _full_),
            pl.BlockSpec((3 * H, 2 * H), _full_),
            pl.BlockSpec((3 * H, H), _full_),
            pl.BlockSpec((1, 3 * H), _full_),
            pl.BlockSpec((1, 3 * H), _full_),
        ],
        out_specs=pl.BlockSpec((R_PAD, H), _full_),
        out_shape=jax.ShapeDtypeStruct((R_PAD, H), jnp.float32),
        scratch_shapes=[
            pltpu.VMEM((R_PAD, H), jnp.float32),
            pltpu.VMEM((R_PAD, 1), jnp.float32),
        ],
    )


def _nblk(nbc):
    return NW * nbc * K // B


_SEGMID14 = _make_segmid(_nblk(NBC[0]))
_SEGFIN3 = _make_segfin(_nblk(NBC[3]))


def kernel(rel_embs, ent_embs, r_to_e_flat, seg_ids, e_r_bias, num_rels,
           W_ih, W_hh, b_ih, b_hh):
    pad = E_PAD - E
    idx_rs = jnp.concatenate(
        [r_to_e_flat.astype(jnp.int32),
         jnp.zeros((pad,), jnp.int32)]).reshape(NW, NB, K)
    seg_rs = jnp.concatenate(
        [seg_ids.astype(jnp.int32),
         jnp.full((pad,), R_PAD - 1, jnp.int32)]).reshape(NW, NB, K)

    b0, b1, b2 = NBC[0], NBC[0] + NBC[1], NBC[0] + NBC[1] + NBC[2]
    idx_c = (idx_rs[:, :b0], idx_rs[:, b0:b1], idx_rs[:, b1:b2],
             idx_rs[:, b2:])
    seg_c = (seg_rs[:, :b0].reshape(_nblk(NBC[0]), 1, B),
             seg_rs[:, b0:b1].reshape(_nblk(NBC[1]), 1, B),
             seg_rs[:, b1:b2].reshape(_nblk(NBC[2]), 1, B),
             seg_rs[:, b2:].reshape(_nblk(NBC[3]), 1, B))

    g1 = _GATHER14(ent_embs, idx_c[0])
    g2 = _GATHER14(ent_embs, idx_c[1])
    g3 = _GATHER14(ent_embs, idx_c[2])
    g4 = _GATHER3(ent_embs, idx_c[3])

    z_s = jnp.zeros((R_PAD, H), jnp.float32)
    z_c = jnp.zeros((R_PAD, 1), jnp.float32)
    s1, c1 = _SEGMID14(seg_c[0], g1.reshape(_nblk(NBC[0]), B, H), z_s, z_c)
    s2, c2 = _SEGMID14(seg_c[1], g2.reshape(_nblk(NBC[1]), B, H), s1, c1)
    s3, c3 = _SEGMID14(seg_c[2], g3.reshape(_nblk(NBC[2]), B, H), s2, c2)
    rel_pad = jnp.pad(rel_embs, ((0, R_PAD - R), (0, 0)))
    out = _SEGFIN3(seg_c[3], g4.reshape(_nblk(NBC[3]), B, H), s3, c3,
                   rel_pad, W_ih, W_hh, b_ih.reshape(1, 3 * H),
                   b_hh.reshape(1, 3 * H))
    return out[:R]
